# baseline (speedup 1.0000x reference)
import functools

import jax
import jax.numpy as jnp
from jax import lax
from jax.experimental import pallas as pl
from jax.experimental.pallas import tpu as pltpu
from jax.experimental.pallas import tpu_sc as plsc

G = 64
NPAD = 10240
NC = 2
NS = 16


def _matmul_call(a, b, BN, bias=None):
    NA, K = a.shape
    KO = b.shape[1]
    NB = NA // BN

    def body(*refs):
        if bias is None:
            a_ref, b_ref, o_ref = refs
            o_ref[...] = jnp.dot(a_ref[...], b_ref[...],
                                 preferred_element_type=jnp.float32)
        else:
            a_ref, b_ref, c_ref, o_ref = refs
            o_ref[...] = jnp.dot(a_ref[...], b_ref[...],
                                 preferred_element_type=jnp.float32) + c_ref[...]

    in_specs = [pl.BlockSpec((BN, K), lambda i: (i, 0)),
                pl.BlockSpec((K, KO), lambda i: (0, 0))]
    args = [a, b]
    if bias is not None:
        in_specs.append(pl.BlockSpec((1, KO), lambda i: (0, 0)))
        args.append(bias)
    return pl.pallas_call(
        body,
        grid=(NB,),
        in_specs=in_specs,
        out_specs=pl.BlockSpec((BN, KO), lambda i: (i, 0)),
        out_shape=jax.ShapeDtypeStruct((NA, KO), jnp.float32),
    )(*args)


def _y2_zero_call(x, W2, BN):
    N, D = x.shape
    H = W2.shape[1]
    NB = N // BN
    BZ = NPAD // 2 // NB

    def body(x_ref, w_ref, y2_ref, z_ref):
        yv = jnp.dot(x_ref[...], w_ref[...],
                     preferred_element_type=jnp.float32)
        y2_ref[...] = jnp.concatenate([yv, yv], axis=1)
        z_ref[...] = jnp.zeros_like(z_ref)

    return pl.pallas_call(
        body,
        grid=(NB,),
        in_specs=[pl.BlockSpec((BN, D), lambda i: (i, 0)),
                  pl.BlockSpec((D, H), lambda i: (0, 0))],
        out_specs=[pl.BlockSpec((BN, 2 * H), lambda i: (i, 0)),
                   pl.BlockSpec((BZ, 2 * H), lambda i: (i, 0))],
        out_shape=[jax.ShapeDtypeStruct((N, 2 * H), jnp.float32),
                   jax.ShapeDtypeStruct((NPAD // 2, 2 * H), jnp.float32)],
    )(x, W2)


@functools.lru_cache(maxsize=None)
def _make_sc_scatter(NCHUNK, H, CH):
    NW = NC * NS
    CPW = NCHUNK // NW
    REM = NCHUNK - CPW * NW
    GL = 6
    LA = 3
    NGL = CPW // GL
    TROWS = NPAD // NS

    mesh = plsc.VectorSubcoreMesh(core_axis_name="c", subcore_axis_name="s")

    @functools.partial(
        pl.kernel,
        mesh=mesh,
        compiler_params=pltpu.CompilerParams(use_tc_tiling_on_sc=False),
        out_type=jax.ShapeDtypeStruct((NC, NPAD, H), jnp.float32),
        scratch_types=[
            pltpu.VMEM((2, GL, CH), jnp.int32),
            pltpu.VMEM((2, GL, CH), jnp.int32),
            pltpu.VMEM((GL, CH, H), jnp.float32),
            pltpu.VMEM_SHARED((NPAD, H), jnp.float32),
            pltpu.SemaphoreType.DMA,
        ],
    )
    def sc_fn(y_hbm, srcx2_hbm, dst_hbm, zero_hbm, out_hbm,
              src_v, dst_v, rows_v, acc_sh, sem_g):
        c = lax.axis_index("c")
        s = lax.axis_index("s")
        tstart = s * TROWS
        pltpu.sync_copy(zero_hbm.at[pl.ds(tstart, TROWS)],
                        acc_sh.at[pl.ds(tstart, TROWS)])
        plsc.subcore_barrier()

        wid = c * NS + s
        row0 = CPW * wid + jnp.minimum(wid, REM)

        def fetch_group(g, ib):
            start = row0 + g * GL
            pltpu.sync_copy(srcx2_hbm.at[pl.ds(start, GL)], src_v.at[ib])
            pltpu.sync_copy(dst_hbm.at[pl.ds(start, GL)], dst_v.at[ib])

        def fire_gather(slot, ib, pos):
            pltpu.async_copy(y_hbm.at[src_v.at[ib, pos]],
                             rows_v.at[slot], sem_g)

        def wait_gather(slot, ib, pos):
            pltpu.make_async_copy(y_hbm.at[src_v.at[ib, pos]],
                                  rows_v.at[slot], sem_g).wait()

        def scatter_sync(slot, ib, pos):
            pltpu.sync_copy(rows_v.at[slot], acc_sh.at[dst_v.at[ib, pos]],
                            add=True)

        fetch_group(0, 0)
        for b in range(LA):
            fire_gather(b, 0, b)

        def body(g, carry):
            ib = lax.rem(g, 2)
            ibn = 1 - ib
            fetch_group(g + 1, ibn)
            for b in range(GL):
                if b + LA < GL:
                    fire_gather(b + LA, ib, b + LA)
                else:
                    fire_gather(b + LA - GL, ibn, b + LA - GL)
                wait_gather(b, ib, b)
                scatter_sync(b, ib, b)
            return carry

        lax.fori_loop(0, NGL - 1, body, 0)
        ibl = (NGL - 1) % 2
        for b in range(GL):
            if b + LA < GL:
                fire_gather(b + LA, ibl, b + LA)
            wait_gather(b, ibl, b)
            scatter_sync(b, ibl, b)

        @pl.when(wid < REM)
        def _tail():
            start = row0 + CPW
            pltpu.sync_copy(srcx2_hbm.at[start], src_v.at[0, 0])
            pltpu.sync_copy(dst_hbm.at[start], dst_v.at[0, 0])
            pltpu.async_copy(y_hbm.at[src_v.at[0, 0]], rows_v.at[0],
                             sem_g).wait()
            pltpu.sync_copy(rows_v.at[0], acc_sh.at[dst_v.at[0, 0]], add=True)

        plsc.subcore_barrier()
        pltpu.sync_copy(acc_sh.at[pl.ds(tstart, TROWS)],
                        out_hbm.at[c, pl.ds(tstart, TROWS)])

    return sc_fn


def _head_call(xw1b, msgp, batch_e, batch_o, Wout, bout,
               D1W, D1b, D2W, D2b, D3W, D3b, BP):
    NP = xw1b.shape[0]
    H = Wout.shape[0]
    M = Wout.shape[1]
    HN = D1W.shape[1]
    T = D3W.shape[1]
    NB = NP // BP

    def body(xw1b_ref, msg_ref, be_ref, bo_ref, wout_ref, bout_ref,
             d1w_ref, d1b_ref, d2w_ref, d2b_ref, d3w_ref, d3b_ref,
             out_ref, acc_ref, cnt_ref):
        i = pl.program_id(0)

        @pl.when(i == 0)
        def _init():
            acc_ref[...] = jnp.zeros_like(acc_ref)
            cnt_ref[...] = jnp.zeros_like(cnt_ref)

        h = jnp.tanh(xw1b_ref[...] + msg_ref[0] + msg_ref[1])
        oh_e = (be_ref[0] == lax.broadcasted_iota(jnp.int32, (G, BP), 0)
                ).astype(jnp.float32)
        oh_o = (bo_ref[0] == lax.broadcasted_iota(jnp.int32, (G, BP), 0)
                ).astype(jnp.float32)
        acc_ref[...] += (
            jnp.dot(oh_e, h[:, :H], preferred_element_type=jnp.float32)
            + jnp.dot(oh_o, h[:, H:], preferred_element_type=jnp.float32))
        cnt_ref[...] += (jnp.sum(oh_e, axis=1, keepdims=True)
                         + jnp.sum(oh_o, axis=1, keepdims=True))

        @pl.when(i == NB - 1)
        def _final():
            pooled = acc_ref[...] / jnp.maximum(cnt_ref[...], 1.0)
            psi = jnp.dot(pooled, wout_ref[...],
                          preferred_element_type=jnp.float32) + bout_ref[...]
            h1 = jnp.dot(jax.nn.relu(psi), d1w_ref[...],
                         preferred_element_type=jnp.float32) + d1b_ref[...]
            h2 = jnp.dot(jax.nn.relu(h1), d2w_ref[...],
                         preferred_element_type=jnp.float32) + d2b_ref[...]
            out_ref[...] = jnp.dot(jax.nn.relu(h2), d3w_ref[...],
                                   preferred_element_type=jnp.float32) + d3b_ref[...]

    full = lambda shape: pl.BlockSpec(shape, lambda i: tuple(0 for _ in shape))
    return pl.pallas_call(
        body,
        grid=(NB,),
        in_specs=[
            pl.BlockSpec((BP, 2 * H), lambda i: (i, 0)),
            pl.BlockSpec((NC, BP, 2 * H), lambda i: (0, i, 0)),
            pl.BlockSpec((1, 1, BP), lambda i: (i, 0, 0)),
            pl.BlockSpec((1, 1, BP), lambda i: (i, 0, 0)),
            full((H, M)), full((1, M)),
            full((M, HN)), full((1, HN)), full((HN, HN)), full((1, HN)),
            full((HN, T)), full((1, T)),
        ],
        out_specs=pl.BlockSpec((G, T), lambda i: (0, 0)),
        out_shape=jax.ShapeDtypeStruct((G, T), jnp.float32),
        scratch_shapes=[pltpu.VMEM((G, H), jnp.float32),
                        pltpu.VMEM((G, H), jnp.float32)],
    )(xw1b, msgp, batch_e, batch_o, Wout, bout,
      D1W, D1b, D2W, D2b, D3W, D3b)


def kernel(x, edge_index, batch, W1, b1, W2, Wout, bout,
           D1W, D1b, D2W, D2b, D3W, D3b):
    N, D = x.shape
    H = W1.shape[1]
    E = edge_index.shape[1]
    CH = 128

    y2, zpair = _y2_zero_call(x, W2, 1000)
    yv = y2.reshape(2 * N, H)
    zero = zpair.reshape(NPAD, H)

    srcx2 = (edge_index[0] * 2).reshape(E // CH, CH)
    dst2d = edge_index[1].reshape(E // CH, CH)
    msg = _make_sc_scatter(E // CH, H, CH)(yv, srcx2, dst2d, zero)
    msgp = msg.reshape(NC, NPAD // 2, 2 * H)

    x2 = x.reshape(N // 2, 2 * D)
    W1blk = jnp.concatenate(
        [jnp.concatenate([W1, jnp.zeros_like(W1)], axis=1),
         jnp.concatenate([jnp.zeros_like(W1), W1], axis=1)], axis=0)
    b1c = jnp.concatenate([b1, b1]).reshape(1, 2 * H)
    xw1b = _matmul_call(x2, W1blk, 1000, bias=b1c)

    b2 = batch.reshape(N // 2, 2)
    NBP = N // 2 // 1000
    batch_e = b2[:, 0].reshape(NBP, 1, 1000)
    batch_o = b2[:, 1].reshape(NBP, 1, 1000)
    return _head_call(xw1b, msgp, batch_e, batch_o, Wout,
                      bout.reshape(1, -1), D1W, D1b.reshape(1, -1),
                      D2W, D2b.reshape(1, -1), D3W, D3b.reshape(1, -1), 1000)


# speedup vs baseline: 1.1065x; 1.1065x over previous
import functools

import jax
import jax.numpy as jnp
from jax import lax
from jax.experimental import pallas as pl
from jax.experimental.pallas import tpu as pltpu
from jax.experimental.pallas import tpu_sc as plsc

G = 64
NPAD = 10240
NC = 2
NS = 16


def _matmul_call(a, b, BN, bias=None):
    NA, K = a.shape
    KO = b.shape[1]
    NB = NA // BN

    def body(*refs):
        if bias is None:
            a_ref, b_ref, o_ref = refs
            o_ref[...] = jnp.dot(a_ref[...], b_ref[...],
                                 preferred_element_type=jnp.float32)
        else:
            a_ref, b_ref, c_ref, o_ref = refs
            o_ref[...] = jnp.dot(a_ref[...], b_ref[...],
                                 preferred_element_type=jnp.float32) + c_ref[...]

    in_specs = [pl.BlockSpec((BN, K), lambda i: (i, 0)),
                pl.BlockSpec((K, KO), lambda i: (0, 0))]
    args = [a, b]
    if bias is not None:
        in_specs.append(pl.BlockSpec((1, KO), lambda i: (0, 0)))
        args.append(bias)
    return pl.pallas_call(
        body,
        grid=(NB,),
        in_specs=in_specs,
        out_specs=pl.BlockSpec((BN, KO), lambda i: (i, 0)),
        out_shape=jax.ShapeDtypeStruct((NA, KO), jnp.float32),
    )(*args)


def _y2_zero_call(x, W2, BN):
    N, D = x.shape
    H = W2.shape[1]
    NB = N // BN
    BZ = NPAD // 2 // NB

    def body(x_ref, w_ref, y2_ref, z_ref):
        yv = jnp.dot(x_ref[...], w_ref[...],
                     preferred_element_type=jnp.float32)
        y2_ref[...] = jnp.concatenate([yv, yv], axis=1)
        z_ref[...] = jnp.zeros_like(z_ref)

    return pl.pallas_call(
        body,
        grid=(NB,),
        in_specs=[pl.BlockSpec((BN, D), lambda i: (i, 0)),
                  pl.BlockSpec((D, H), lambda i: (0, 0))],
        out_specs=[pl.BlockSpec((BN, 2 * H), lambda i: (i, 0)),
                   pl.BlockSpec((BZ, 2 * H), lambda i: (i, 0))],
        out_shape=[jax.ShapeDtypeStruct((N, 2 * H), jnp.float32),
                   jax.ShapeDtypeStruct((NPAD // 2, 2 * H), jnp.float32)],
    )(x, W2)


@functools.lru_cache(maxsize=None)
def _make_sc_scatter(NCHUNK, H, CH):
    NW = NC * NS
    CPW = NCHUNK // NW
    REM = NCHUNK - CPW * NW
    GL = 6
    LA = 3
    NGL = CPW // GL
    TROWS = NPAD // NS

    mesh = plsc.VectorSubcoreMesh(core_axis_name="c", subcore_axis_name="s")

    @functools.partial(
        pl.kernel,
        mesh=mesh,
        compiler_params=pltpu.CompilerParams(use_tc_tiling_on_sc=False),
        out_type=jax.ShapeDtypeStruct((NC, NPAD, H), jnp.float32),
        scratch_types=[
            pltpu.VMEM((2, GL, CH), jnp.int32),
            pltpu.VMEM((2, GL, CH), jnp.int32),
            pltpu.VMEM((GL, CH, H), jnp.float32),
            pltpu.VMEM_SHARED((NPAD, H), jnp.float32),
            pltpu.SemaphoreType.DMA,
        ],
    )
    def sc_fn(y_hbm, ei_hbm, zero_hbm, out_hbm,
              src_v, dst_v, rows_v, acc_sh, sem_g):
        c = lax.axis_index("c")
        s = lax.axis_index("s")
        tstart = s * TROWS
        pltpu.sync_copy(zero_hbm.at[pl.ds(tstart, TROWS)],
                        acc_sh.at[pl.ds(tstart, TROWS)])
        plsc.subcore_barrier()

        wid = c * NS + s
        row0 = CPW * wid + jnp.minimum(wid, REM)

        def fetch_group(g, ib):
            start = row0 + g * GL
            pltpu.sync_copy(ei_hbm.at[0, pl.ds(start, GL)], src_v.at[ib])
            pltpu.sync_copy(ei_hbm.at[1, pl.ds(start, GL)], dst_v.at[ib])
            for p in range(GL):
                for k in range(CH // 16):
                    sl = pl.ds(k * 16, 16)
                    src_v[ib, p, sl] = src_v[ib, p, sl] * 2

        def fire_gather(slot, ib, pos):
            pltpu.async_copy(y_hbm.at[src_v.at[ib, pos]],
                             rows_v.at[slot], sem_g)

        def wait_gather(slot, ib, pos):
            pltpu.make_async_copy(y_hbm.at[src_v.at[ib, pos]],
                                  rows_v.at[slot], sem_g).wait()

        def scatter_sync(slot, ib, pos):
            pltpu.sync_copy(rows_v.at[slot], acc_sh.at[dst_v.at[ib, pos]],
                            add=True)

        fetch_group(0, 0)
        for b in range(LA):
            fire_gather(b, 0, b)

        def body(g, carry):
            ib = lax.rem(g, 2)
            ibn = 1 - ib
            fetch_group(g + 1, ibn)
            for b in range(GL):
                if b + LA < GL:
                    fire_gather(b + LA, ib, b + LA)
                else:
                    fire_gather(b + LA - GL, ibn, b + LA - GL)
                wait_gather(b, ib, b)
                scatter_sync(b, ib, b)
            return carry

        lax.fori_loop(0, NGL - 1, body, 0)
        ibl = (NGL - 1) % 2
        for b in range(GL):
            if b + LA < GL:
                fire_gather(b + LA, ibl, b + LA)
            wait_gather(b, ibl, b)
            scatter_sync(b, ibl, b)

        @pl.when(wid < REM)
        def _tail():
            start = row0 + CPW
            pltpu.sync_copy(ei_hbm.at[0, start], src_v.at[0, 0])
            pltpu.sync_copy(ei_hbm.at[1, start], dst_v.at[0, 0])
            for k in range(CH // 16):
                sl = pl.ds(k * 16, 16)
                src_v[0, 0, sl] = src_v[0, 0, sl] * 2
            pltpu.async_copy(y_hbm.at[src_v.at[0, 0]], rows_v.at[0],
                             sem_g).wait()
            pltpu.sync_copy(rows_v.at[0], acc_sh.at[dst_v.at[0, 0]], add=True)

        plsc.subcore_barrier()
        pltpu.sync_copy(acc_sh.at[pl.ds(tstart, TROWS)],
                        out_hbm.at[c, pl.ds(tstart, TROWS)])

    return sc_fn


def _head_call(xw1b, msgp, batch_e, batch_o, Wout, bout,
               D1W, D1b, D2W, D2b, D3W, D3b, BP):
    NP = xw1b.shape[0]
    H = Wout.shape[0]
    M = Wout.shape[1]
    HN = D1W.shape[1]
    T = D3W.shape[1]
    NB = NP // BP

    def body(xw1b_ref, msg_ref, be_ref, bo_ref, wout_ref, bout_ref,
             d1w_ref, d1b_ref, d2w_ref, d2b_ref, d3w_ref, d3b_ref,
             out_ref, acc_ref, cnt_ref):
        i = pl.program_id(0)

        @pl.when(i == 0)
        def _init():
            acc_ref[...] = jnp.zeros_like(acc_ref)
            cnt_ref[...] = jnp.zeros_like(cnt_ref)

        h = jnp.tanh(xw1b_ref[...] + msg_ref[0] + msg_ref[1])
        oh_e = (be_ref[0] == lax.broadcasted_iota(jnp.int32, (G, BP), 0)
                ).astype(jnp.float32)
        oh_o = (bo_ref[0] == lax.broadcasted_iota(jnp.int32, (G, BP), 0)
                ).astype(jnp.float32)
        acc_ref[...] += (
            jnp.dot(oh_e, h[:, :H], preferred_element_type=jnp.float32)
            + jnp.dot(oh_o, h[:, H:], preferred_element_type=jnp.float32))
        cnt_ref[...] += (jnp.sum(oh_e, axis=1, keepdims=True)
                         + jnp.sum(oh_o, axis=1, keepdims=True))

        @pl.when(i == NB - 1)
        def _final():
            pooled = acc_ref[...] / jnp.maximum(cnt_ref[...], 1.0)
            psi = jnp.dot(pooled, wout_ref[...],
                          preferred_element_type=jnp.float32) + bout_ref[...]
            h1 = jnp.dot(jax.nn.relu(psi), d1w_ref[...],
                         preferred_element_type=jnp.float32) + d1b_ref[...]
            h2 = jnp.dot(jax.nn.relu(h1), d2w_ref[...],
                         preferred_element_type=jnp.float32) + d2b_ref[...]
            out_ref[...] = jnp.dot(jax.nn.relu(h2), d3w_ref[...],
                                   preferred_element_type=jnp.float32) + d3b_ref[...]

    full = lambda shape: pl.BlockSpec(shape, lambda i: tuple(0 for _ in shape))
    return pl.pallas_call(
        body,
        grid=(NB,),
        in_specs=[
            pl.BlockSpec((BP, 2 * H), lambda i: (i, 0)),
            pl.BlockSpec((NC, BP, 2 * H), lambda i: (0, i, 0)),
            pl.BlockSpec((1, 1, BP), lambda i: (i, 0, 0)),
            pl.BlockSpec((1, 1, BP), lambda i: (i, 0, 0)),
            full((H, M)), full((1, M)),
            full((M, HN)), full((1, HN)), full((HN, HN)), full((1, HN)),
            full((HN, T)), full((1, T)),
        ],
        out_specs=pl.BlockSpec((G, T), lambda i: (0, 0)),
        out_shape=jax.ShapeDtypeStruct((G, T), jnp.float32),
        scratch_shapes=[pltpu.VMEM((G, H), jnp.float32),
                        pltpu.VMEM((G, H), jnp.float32)],
    )(xw1b, msgp, batch_e, batch_o, Wout, bout,
      D1W, D1b, D2W, D2b, D3W, D3b)


def kernel(x, edge_index, batch, W1, b1, W2, Wout, bout,
           D1W, D1b, D2W, D2b, D3W, D3b):
    N, D = x.shape
    H = W1.shape[1]
    E = edge_index.shape[1]
    CH = 128

    y2, zpair = _y2_zero_call(x, W2, 1000)
    yv = y2.reshape(2 * N, H)
    zero = zpair.reshape(NPAD, H)

    ei3 = edge_index.reshape(2, E // CH, CH)
    msg = _make_sc_scatter(E // CH, H, CH)(yv, ei3, zero)
    msgp = msg.reshape(NC, NPAD // 2, 2 * H)

    x2 = x.reshape(N // 2, 2 * D)
    W1blk = jnp.concatenate(
        [jnp.concatenate([W1, jnp.zeros_like(W1)], axis=1),
         jnp.concatenate([jnp.zeros_like(W1), W1], axis=1)], axis=0)
    b1c = jnp.concatenate([b1, b1]).reshape(1, 2 * H)
    xw1b = _matmul_call(x2, W1blk, 1000, bias=b1c)

    b2 = batch.reshape(N // 2, 2)
    NBP = N // 2 // 1000
    batch_e = b2[:, 0].reshape(NBP, 1, 1000)
    batch_o = b2[:, 1].reshape(NBP, 1, 1000)
    return _head_call(xw1b, msgp, batch_e, batch_o, Wout,
                      bout.reshape(1, -1), D1W, D1b.reshape(1, -1),
                      D2W, D2b.reshape(1, -1), D3W, D3b.reshape(1, -1), 1000)


# speedup vs baseline: 1.1091x; 1.0024x over previous
import functools

import jax
import jax.numpy as jnp
from jax import lax
from jax.experimental import pallas as pl
from jax.experimental.pallas import tpu as pltpu
from jax.experimental.pallas import tpu_sc as plsc

G = 64
NPAD = 10240
NC = 2
NS = 16


def _matmul_call(a, b, BN, bias=None):
    NA, K = a.shape
    KO = b.shape[1]
    NB = NA // BN

    def body(*refs):
        if bias is None:
            a_ref, b_ref, o_ref = refs
            o_ref[...] = jnp.dot(a_ref[...], b_ref[...],
                                 preferred_element_type=jnp.float32)
        else:
            a_ref, b_ref, c_ref, o_ref = refs
            o_ref[...] = jnp.dot(a_ref[...], b_ref[...],
                                 preferred_element_type=jnp.float32) + c_ref[...]

    in_specs = [pl.BlockSpec((BN, K), lambda i: (i, 0)),
                pl.BlockSpec((K, KO), lambda i: (0, 0))]
    args = [a, b]
    if bias is not None:
        in_specs.append(pl.BlockSpec((1, KO), lambda i: (0, 0)))
        args.append(bias)
    return pl.pallas_call(
        body,
        grid=(NB,),
        in_specs=in_specs,
        out_specs=pl.BlockSpec((BN, KO), lambda i: (i, 0)),
        out_shape=jax.ShapeDtypeStruct((NA, KO), jnp.float32),
    )(*args)


def _y2_zero_call(x, W2, BN):
    N, D = x.shape
    H = W2.shape[1]
    NB = N // BN
    BZ = NPAD // 2 // NB

    def body(x_ref, w_ref, y2_ref, z_ref):
        yv = jnp.dot(x_ref[...], w_ref[...],
                     preferred_element_type=jnp.float32)
        y2_ref[...] = jnp.concatenate([yv, yv], axis=1)
        z_ref[...] = jnp.zeros_like(z_ref)

    return pl.pallas_call(
        body,
        grid=(NB,),
        in_specs=[pl.BlockSpec((BN, D), lambda i: (i, 0)),
                  pl.BlockSpec((D, H), lambda i: (0, 0))],
        out_specs=[pl.BlockSpec((BN, 2 * H), lambda i: (i, 0)),
                   pl.BlockSpec((BZ, 2 * H), lambda i: (i, 0))],
        out_shape=[jax.ShapeDtypeStruct((N, 2 * H), jnp.float32),
                   jax.ShapeDtypeStruct((NPAD // 2, 2 * H), jnp.float32)],
    )(x, W2)


@functools.lru_cache(maxsize=None)
def _make_sc_scatter(NCHUNK, H, CH):
    NW = NC * NS
    CPW = NCHUNK // NW
    REM = NCHUNK - CPW * NW
    GL = 6
    LA = 4
    NGL = CPW // GL
    TROWS = NPAD // NS

    mesh = plsc.VectorSubcoreMesh(core_axis_name="c", subcore_axis_name="s")

    @functools.partial(
        pl.kernel,
        mesh=mesh,
        compiler_params=pltpu.CompilerParams(use_tc_tiling_on_sc=False),
        out_type=jax.ShapeDtypeStruct((NC, NPAD, H), jnp.float32),
        scratch_types=[
            pltpu.VMEM((2, GL, CH), jnp.int32),
            pltpu.VMEM((2, GL, CH), jnp.int32),
            pltpu.VMEM((GL, CH, H), jnp.float32),
            pltpu.VMEM_SHARED((NPAD, H), jnp.float32),
            pltpu.SemaphoreType.DMA,
        ],
    )
    def sc_fn(y_hbm, ei_hbm, zero_hbm, out_hbm,
              src_v, dst_v, rows_v, acc_sh, sem_g):
        c = lax.axis_index("c")
        s = lax.axis_index("s")
        tstart = s * TROWS
        pltpu.sync_copy(zero_hbm.at[pl.ds(tstart, TROWS)],
                        acc_sh.at[pl.ds(tstart, TROWS)])
        plsc.subcore_barrier()

        wid = c * NS + s
        row0 = CPW * wid + jnp.minimum(wid, REM)

        def fetch_group(g, ib):
            start = row0 + g * GL
            pltpu.sync_copy(ei_hbm.at[0, pl.ds(start, GL)], src_v.at[ib])
            pltpu.sync_copy(ei_hbm.at[1, pl.ds(start, GL)], dst_v.at[ib])
            for p in range(GL):
                for k in range(CH // 16):
                    sl = pl.ds(k * 16, 16)
                    src_v[ib, p, sl] = src_v[ib, p, sl] * 2

        def fire_gather(slot, ib, pos):
            pltpu.async_copy(y_hbm.at[src_v.at[ib, pos]],
                             rows_v.at[slot], sem_g)

        def wait_gather(slot, ib, pos):
            pltpu.make_async_copy(y_hbm.at[src_v.at[ib, pos]],
                                  rows_v.at[slot], sem_g).wait()

        def scatter_sync(slot, ib, pos):
            pltpu.sync_copy(rows_v.at[slot], acc_sh.at[dst_v.at[ib, pos]],
                            add=True)

        fetch_group(0, 0)
        for b in range(LA):
            fire_gather(b, 0, b)

        def body(g, carry):
            ib = lax.rem(g, 2)
            ibn = 1 - ib
            fetch_group(g + 1, ibn)
            for b in range(GL):
                if b + LA < GL:
                    fire_gather(b + LA, ib, b + LA)
                else:
                    fire_gather(b + LA - GL, ibn, b + LA - GL)
                wait_gather(b, ib, b)
                scatter_sync(b, ib, b)
            return carry

        lax.fori_loop(0, NGL - 1, body, 0)
        ibl = (NGL - 1) % 2
        for b in range(GL):
            if b + LA < GL:
                fire_gather(b + LA, ibl, b + LA)
            wait_gather(b, ibl, b)
            scatter_sync(b, ibl, b)

        @pl.when(wid < REM)
        def _tail():
            start = row0 + CPW
            pltpu.sync_copy(ei_hbm.at[0, start], src_v.at[0, 0])
            pltpu.sync_copy(ei_hbm.at[1, start], dst_v.at[0, 0])
            for k in range(CH // 16):
                sl = pl.ds(k * 16, 16)
                src_v[0, 0, sl] = src_v[0, 0, sl] * 2
            pltpu.async_copy(y_hbm.at[src_v.at[0, 0]], rows_v.at[0],
                             sem_g).wait()
            pltpu.sync_copy(rows_v.at[0], acc_sh.at[dst_v.at[0, 0]], add=True)

        plsc.subcore_barrier()
        pltpu.sync_copy(acc_sh.at[pl.ds(tstart, TROWS)],
                        out_hbm.at[c, pl.ds(tstart, TROWS)])

    return sc_fn


def _head_call(xw1b, msgp, batch_e, batch_o, Wout, bout,
               D1W, D1b, D2W, D2b, D3W, D3b, BP):
    NP = xw1b.shape[0]
    H = Wout.shape[0]
    M = Wout.shape[1]
    HN = D1W.shape[1]
    T = D3W.shape[1]
    NB = NP // BP

    def body(xw1b_ref, msg_ref, be_ref, bo_ref, wout_ref, bout_ref,
             d1w_ref, d1b_ref, d2w_ref, d2b_ref, d3w_ref, d3b_ref,
             out_ref, acc_ref, cnt_ref):
        i = pl.program_id(0)

        @pl.when(i == 0)
        def _init():
            acc_ref[...] = jnp.zeros_like(acc_ref)
            cnt_ref[...] = jnp.zeros_like(cnt_ref)

        h = jnp.tanh(xw1b_ref[...] + msg_ref[0] + msg_ref[1])
        oh_e = (be_ref[0] == lax.broadcasted_iota(jnp.int32, (G, BP), 0)
                ).astype(jnp.float32)
        oh_o = (bo_ref[0] == lax.broadcasted_iota(jnp.int32, (G, BP), 0)
                ).astype(jnp.float32)
        acc_ref[...] += (
            jnp.dot(oh_e, h[:, :H], preferred_element_type=jnp.float32)
            + jnp.dot(oh_o, h[:, H:], preferred_element_type=jnp.float32))
        cnt_ref[...] += (jnp.sum(oh_e, axis=1, keepdims=True)
                         + jnp.sum(oh_o, axis=1, keepdims=True))

        @pl.when(i == NB - 1)
        def _final():
            pooled = acc_ref[...] / jnp.maximum(cnt_ref[...], 1.0)
            psi = jnp.dot(pooled, wout_ref[...],
                          preferred_element_type=jnp.float32) + bout_ref[...]
            h1 = jnp.dot(jax.nn.relu(psi), d1w_ref[...],
                         preferred_element_type=jnp.float32) + d1b_ref[...]
            h2 = jnp.dot(jax.nn.relu(h1), d2w_ref[...],
                         preferred_element_type=jnp.float32) + d2b_ref[...]
            out_ref[...] = jnp.dot(jax.nn.relu(h2), d3w_ref[...],
                                   preferred_element_type=jnp.float32) + d3b_ref[...]

    full = lambda shape: pl.BlockSpec(shape, lambda i: tuple(0 for _ in shape))
    return pl.pallas_call(
        body,
        grid=(NB,),
        in_specs=[
            pl.BlockSpec((BP, 2 * H), lambda i: (i, 0)),
            pl.BlockSpec((NC, BP, 2 * H), lambda i: (0, i, 0)),
            pl.BlockSpec((1, 1, BP), lambda i: (i, 0, 0)),
            pl.BlockSpec((1, 1, BP), lambda i: (i, 0, 0)),
            full((H, M)), full((1, M)),
            full((M, HN)), full((1, HN)), full((HN, HN)), full((1, HN)),
            full((HN, T)), full((1, T)),
        ],
        out_specs=pl.BlockSpec((G, T), lambda i: (0, 0)),
        out_shape=jax.ShapeDtypeStruct((G, T), jnp.float32),
        scratch_shapes=[pltpu.VMEM((G, H), jnp.float32),
                        pltpu.VMEM((G, H), jnp.float32)],
    )(xw1b, msgp, batch_e, batch_o, Wout, bout,
      D1W, D1b, D2W, D2b, D3W, D3b)


def kernel(x, edge_index, batch, W1, b1, W2, Wout, bout,
           D1W, D1b, D2W, D2b, D3W, D3b):
    N, D = x.shape
    H = W1.shape[1]
    E = edge_index.shape[1]
    CH = 128

    y2, zpair = _y2_zero_call(x, W2, 1000)
    yv = y2.reshape(2 * N, H)
    zero = zpair.reshape(NPAD, H)

    ei3 = edge_index.reshape(2, E // CH, CH)
    msg = _make_sc_scatter(E // CH, H, CH)(yv, ei3, zero)
    msgp = msg.reshape(NC, NPAD // 2, 2 * H)

    x2 = x.reshape(N // 2, 2 * D)
    W1blk = jnp.concatenate(
        [jnp.concatenate([W1, jnp.zeros_like(W1)], axis=1),
         jnp.concatenate([jnp.zeros_like(W1), W1], axis=1)], axis=0)
    b1c = jnp.concatenate([b1, b1]).reshape(1, 2 * H)
    xw1b = _matmul_call(x2, W1blk, 1000, bias=b1c)

    b2 = batch.reshape(N // 2, 2)
    NBP = N // 2 // 1000
    batch_e = b2[:, 0].reshape(NBP, 1, 1000)
    batch_o = b2[:, 1].reshape(NBP, 1, 1000)
    return _head_call(xw1b, msgp, batch_e, batch_o, Wout,
                      bout.reshape(1, -1), D1W, D1b.reshape(1, -1),
                      D2W, D2b.reshape(1, -1), D3W, D3b.reshape(1, -1), 1000)
